# baseline (speedup 1.0000x reference)
import functools

import jax
import jax.numpy as jnp
from jax import lax
from jax.experimental import pallas as pl
from jax.experimental.pallas import tpu as pltpu
from jax.experimental.pallas import tpu_sc as plsc

_EMB_DIM = 64
_BATCH = 16384
_NC = 2
_NS = 16
_NW = _NC * _NS
_B_PER_W = _BATCH // _NW
_L = 16
_NG = _B_PER_W // _L

_mesh = plsc.VectorSubcoreMesh(core_axis_name="c", subcore_axis_name="s")


@functools.partial(
    pl.kernel,
    mesh=_mesh,
    out_type=jax.ShapeDtypeStruct((_BATCH * _EMB_DIM,), jnp.float32),
    scratch_types=[
        pltpu.VMEM((_B_PER_W,), jnp.int32),
        pltpu.VMEM((2, _L * 8, _EMB_DIM), jnp.float32),
        pltpu.VMEM((_B_PER_W * _EMB_DIM,), jnp.float32),
        pltpu.SemaphoreType.DMA,
    ],
    compiler_params=pltpu.CompilerParams(needs_layout_passes=False),
)
def _gather_kernel(idx_hbm, table_hbm, out_hbm, idx_v, tiles_v, out_v, sem):
    wid = lax.axis_index("s") * _NC + lax.axis_index("c")
    base = wid * _B_PER_W
    pltpu.sync_copy(idx_hbm.at[pl.ds(base, _B_PER_W)], idx_v)
    lane = lax.iota(jnp.int32, 16)

    def fire_group(g, buf):
        idxg = idx_v[pl.ds(g * _L, _L)]
        t_vec = lax.bitwise_and(idxg, ~7)
        for j in range(_L):
            t_s = pl.multiple_of(jnp.max(jnp.where(lane == j, t_vec, 0)), 8)
            pltpu.async_copy(
                table_hbm.at[pl.ds(t_s, 8)], tiles_v.at[buf, pl.ds(j * 8, 8)], sem
            )

    def drain_and_extract(g, buf):
        idxg = idx_v[pl.ds(g * _L, _L)]
        r_vec = lax.bitwise_and(idxg, 7)
        pltpu.make_async_copy(
            table_hbm.at[pl.ds(0, _L * 8)], tiles_v.at[buf], sem
        ).wait()
        for j in range(_L):
            r_s = jnp.max(jnp.where(lane == j, r_vec, 0))
            pos = g * (_L * _EMB_DIM) + j * _EMB_DIM
            for c in range(_EMB_DIM // _L):
                v = tiles_v[buf, j * 8 + r_s, pl.ds(c * _L, _L)]
                out_v[pl.ds(pos + c * _L, _L)] = v

    fire_group(0, 0)

    def body(g, carry):
        buf = lax.rem(g, 2)
        nbuf = lax.rem(g + 1, 2)

        @pl.when(g < _NG - 1)
        def _fire_next():
            fire_group(g + 1, nbuf)

        drain_and_extract(g, buf)
        return carry

    lax.fori_loop(0, _NG, body, 0)
    pltpu.sync_copy(out_v, out_hbm.at[pl.ds(base * _EMB_DIM, _B_PER_W * _EMB_DIM)])


def kernel(input, table):
    out = _gather_kernel(input.astype(jnp.int32), table)
    return out.reshape(1, 1, -1)


# speedup vs baseline: 1.5398x; 1.5398x over previous
import functools

import jax
import jax.numpy as jnp
from jax import lax
from jax.experimental import pallas as pl
from jax.experimental.pallas import tpu as pltpu
from jax.experimental.pallas import tpu_sc as plsc

_EMB_DIM = 64
_BATCH = 16384
_NC = 2
_NS = 16
_NW = _NC * _NS
_B_PER_W = _BATCH // _NW
_L = 16
_NG = _B_PER_W // _L

_mesh = plsc.VectorSubcoreMesh(core_axis_name="c", subcore_axis_name="s")


@functools.partial(
    pl.kernel,
    mesh=_mesh,
    out_type=jax.ShapeDtypeStruct((_BATCH, _EMB_DIM), jnp.float32),
    scratch_types=[
        pltpu.VMEM((_B_PER_W,), jnp.int32),
        pltpu.VMEM((_B_PER_W, _EMB_DIM), jnp.float32),
        pltpu.SemaphoreType.DMA,
    ],
    compiler_params=pltpu.CompilerParams(needs_layout_passes=False),
)
def _gather_kernel(idx_hbm, table_hbm, out_hbm, idx_v, out_v, sem):
    wid = lax.axis_index("s") * _NC + lax.axis_index("c")
    base = wid * _B_PER_W
    pltpu.sync_copy(idx_hbm.at[pl.ds(base, _B_PER_W)], idx_v)
    lane = lax.iota(jnp.int32, 16)

    def body(g, carry):
        idxg = idx_v[pl.ds(g * _L, _L)]
        for j in range(_L):
            row_s = jnp.max(jnp.where(lane == j, idxg, 0))
            pltpu.async_copy(table_hbm.at[row_s, 0], out_v.at[g * _L + j], sem)
        return carry

    lax.fori_loop(0, _NG, body, 0)
    pltpu.make_async_copy(out_hbm.at[pl.ds(0, _B_PER_W)], out_v, sem).wait()
    pltpu.sync_copy(out_v, out_hbm.at[pl.ds(base, _B_PER_W)])


def kernel(input, table):
    table3 = table.reshape(1000000, 1, _EMB_DIM)
    out = _gather_kernel(input.astype(jnp.int32), table3)
    return out.reshape(1, 1, -1)
